# baseline (speedup 1.0000x reference)
import functools

import jax
import jax.numpy as jnp
import numpy as np
from jax.experimental import pallas as pl
from jax.experimental.pallas import tpu as pltpu

B, C, T = 32, 16, 4096
DIM, DIM2, E, NTASKS, KTOP = 128, 256, 8, 5, 2
NP = 18
NTOK = 6
KC = 13
CHUNKS = 8
CK = (DIM2 * DIM) // CHUNKS
SQRT2 = np.float32(np.sqrt(2.0))


def _gelu(v):
    return v * 0.5 * (1.0 + jax.lax.erf(v / SQRT2))


def _ln_rows(v, s, b, eps=1e-5):
    m = jnp.mean(v, axis=-1, keepdims=True)
    var = jnp.mean((v - m) ** 2, axis=-1, keepdims=True)
    return (v - m) * jax.lax.rsqrt(var + eps) * s + b


def _body(xin_ref, w0_ref, tok_b_ref, cls_ref, tls_ref, tlb_ref,
          arep_ref, b1rep_ref, w2f_ref,
          ds_b2_ref, dls_ref, dlb_ref,
          temb_ref, tid_ref, gw_ref, gb_ref,
          ewt_ref, eb_ref, uwt_ref, ub_ref, ols_ref, olb_ref,
          out_ref, h7_ref, out1_ref, o2_ref):
    i = pl.program_id(0)

    @pl.when(i == 0)
    def _front():
        X = xin_ref[:]
        w0 = w0_ref[:]
        tb = tok_b_ref[:]
        tls, tlb = tls_ref[:], tlb_ref[:]
        cls_ln = _ln_rows(cls_ref[:], tls, tlb)
        h7_ref[0] = jnp.broadcast_to(cls_ln, (B, DIM))
        for m in range(NTOK):
            t = None
            for p in range(3 * m, 3 * m + 3):
                Xp = X if p == 0 else jnp.concatenate(
                    [X[:, 2 * p:], X[:, :2 * p]], axis=1)
                cp = _gelu(jnp.dot(Xp, w0,
                                   preferred_element_type=jnp.float32) + tb)
                t = cp if t is None else jnp.maximum(t, cp)
            h7_ref[1 + m] = _ln_rows(t, tls, tlb)
        a3 = arep_ref[:]
        out1c = b1rep_ref[:] + h7_ref[0].reshape(B, 1, DIM) * a3[0]
        for n in range(1, 7):
            out1c = out1c + h7_ref[n].reshape(B, 1, DIM) * a3[n]
        out1_ref[:] = out1c.astype(jnp.bfloat16).reshape(B, DIM2 * DIM)

    w2b = w2f_ref[:].astype(jnp.bfloat16).reshape(DIM2 // CHUNKS, DIM2 * DIM)
    o2_ref[i] = jax.lax.dot_general(
        out1_ref[:], w2b, (((1,), (1,)), ((), ())),
        preferred_element_type=jnp.float32)

    @pl.when(i == CHUNKS - 1)
    def _tail():
        out2 = (o2_ref[:].transpose(1, 0, 2).reshape(B, DIM2)
                + ds_b2_ref[:])
        g0 = _gelu(_ln_rows(out2, dls_ref[:], dlb_ref[:]))
        ids = tid_ref[:]
        onehot = (jax.lax.broadcasted_iota(jnp.int32, (B, NTASKS), 1)
                  == ids).astype(jnp.float32)
        t_vec = jnp.dot(onehot, temb_ref[:],
                        preferred_element_type=jnp.float32)
        dims_t = (((1,), (1,)), ((), ()))
        gw = gw_ref[:]
        logits = (jax.lax.dot_general(g0, gw[:, :DIM2], dims_t,
                                      preferred_element_type=jnp.float32)
                  + jax.lax.dot_general(t_vec, gw[:, DIM2:], dims_t,
                                        preferred_element_type=jnp.float32)
                  + gb_ref[:])
        idx = jax.lax.broadcasted_iota(jnp.int32, (B, E), 1)
        m1 = jnp.max(logits, axis=-1, keepdims=True)
        i1 = jnp.min(jnp.where(logits == m1, idx, E), axis=-1, keepdims=True)
        l2 = jnp.where(idx == i1, -jnp.inf, logits)
        m2 = jnp.max(l2, axis=-1, keepdims=True)
        i2 = jnp.min(jnp.where(l2 == m2, idx, E), axis=-1, keepdims=True)
        sel = (idx == i1) | (idx == i2)
        ex = jnp.where(sel, jnp.exp(logits - m1), 0.0)
        gates = ex / jnp.sum(ex, axis=-1, keepdims=True)
        t_task = jnp.zeros((B, DIM2), jnp.float32)
        for e in range(E):
            eo = _gelu(jax.lax.dot_general(
                g0, ewt_ref[e], dims_t, preferred_element_type=jnp.float32)
                + eb_ref[e:e + 1, :])
            t_task = t_task + gates[:, e:e + 1] * eo
        t_univ = _gelu(jax.lax.dot_general(
            g0, uwt_ref[:], dims_t, preferred_element_type=jnp.float32)
            + ub_ref[:])
        omega = 1.0 - jnp.max(gates, axis=-1, keepdims=True)
        out = t_task + omega * t_univ
        out_ref[:] = _ln_rows(out, ols_ref[:], olb_ref[:])


@jax.jit
def kernel(x, tok_conv_w, tok_conv_b, cls_token, tok_ln_s, tok_ln_b,
           ds_w1, ds_b1, ds_w2, ds_b2, ds_ln_s, ds_ln_b,
           task_embed, gate_w, gate_b, expert_w, expert_b,
           univ_w, univ_b, out_ln_s, out_ln_b, task_ids):
    xin = jnp.pad(x[:, :, :2 * (NP - 1) + KC - 6],
                  ((0, 0), (0, 0), (6, 1))).reshape(B, C * 48)
    w0 = jnp.pad(tok_conv_w.reshape(DIM, C, KC).transpose(1, 2, 0),
                 ((0, 0), (0, 48 - KC), (0, 0))).reshape(C * 48, DIM)
    a7t = ds_w1[:, 0, 0, 6:6 + 7].T
    arep = jnp.broadcast_to(a7t[:, :, None], (7, DIM2, DIM))
    b1rep = jnp.broadcast_to(ds_b1[None, :, None], (1, DIM2, DIM))
    w2f = ds_w2.reshape(DIM2, DIM2, DIM)
    row = lambda v: v.reshape(1, -1)

    grid = (CHUNKS,)
    full = lambda shape: pl.BlockSpec(shape, lambda i: (0,) * len(shape))
    out = pl.pallas_call(
        _body,
        grid=grid,
        in_specs=[
            full((B, C * 48)),
            full((C * 48, DIM)),
            full((1, DIM)),
            full((1, DIM)),
            full((1, DIM)),
            full((1, DIM)),
            full((7, DIM2, DIM)),
            full((1, DIM2, DIM)),
            pl.BlockSpec((DIM2 // CHUNKS, DIM2, DIM),
                         lambda i: (i, 0, 0)),
            full((1, DIM2)),
            full((1, DIM2)),
            full((1, DIM2)),
            full((NTASKS, DIM2)),
            full((B, 1)),
            full((E, 2 * DIM2)),
            full((1, E)),
            full((E, DIM2, DIM2)),
            full((E, DIM2)),
            full((DIM2, DIM2)),
            full((1, DIM2)),
            full((1, DIM2)),
            full((1, DIM2)),
        ],
        out_specs=pl.BlockSpec((B, DIM2), lambda i: (0, 0)),
        out_shape=jax.ShapeDtypeStruct((B, DIM2), jnp.float32),
        scratch_shapes=[
            pltpu.VMEM((7, B, DIM), jnp.float32),
            pltpu.VMEM((B, DIM2 * DIM), jnp.bfloat16),
            pltpu.VMEM((CHUNKS, B, DIM2 // CHUNKS), jnp.float32),
        ],
        compiler_params=pltpu.CompilerParams(
            dimension_semantics=("arbitrary",)),
    )(xin, w0, row(tok_conv_b), cls_token[0], row(tok_ln_s), row(tok_ln_b),
      arep, b1rep, w2f,
      row(ds_b2), row(ds_ln_s), row(ds_ln_b),
      task_embed, task_ids.reshape(B, 1),
      gate_w, row(gate_b),
      expert_w, expert_b, univ_w, row(univ_b), row(out_ln_s), row(out_ln_b))
    return out


# speedup vs baseline: 1.1347x; 1.1347x over previous
import functools

import jax
import jax.numpy as jnp
import numpy as np
from jax.experimental import pallas as pl
from jax.experimental.pallas import tpu as pltpu

B, C, T = 32, 16, 4096
DIM, DIM2, E, NTASKS, KTOP = 128, 256, 8, 5, 2
NP = 18
NTOK = 6
KC = 13
CHUNKS = 8
CK = (DIM2 * DIM) // CHUNKS
SQRT2 = np.float32(np.sqrt(2.0))


def _gelu(v):
    return v * 0.5 * (1.0 + jax.lax.erf(v / SQRT2))


def _ln_rows(v, s, b, eps=1e-5):
    m = jnp.mean(v, axis=-1, keepdims=True)
    var = jnp.mean((v - m) ** 2, axis=-1, keepdims=True)
    return (v - m) * jax.lax.rsqrt(var + eps) * s + b


def _body(xin_ref, w0_ref, tok_b_ref, cls_ref, tls_ref, tlb_ref,
          aT_ref, b1c_ref, w2f_ref,
          ds_b2_ref, dls_ref, dlb_ref,
          temb_ref, tid_ref, gw_ref, gb_ref,
          ewt_ref, eb_ref, uwt_ref, ub_ref, ols_ref, olb_ref,
          out_ref, h7_ref, acc_ref):
    i = pl.program_id(0)

    @pl.when(i == 0)
    def _front():
        X = xin_ref[:]
        w0 = w0_ref[:]
        tb = tok_b_ref[:]
        tls, tlb = tls_ref[:], tlb_ref[:]
        cls_ln = _ln_rows(cls_ref[:], tls, tlb)
        h7_ref[0] = jnp.broadcast_to(cls_ln, (B, DIM))
        for m in range(NTOK):
            t = None
            for p in range(3 * m, 3 * m + 3):
                Xp = X if p == 0 else jnp.concatenate(
                    [X[:, 2 * p:], X[:, :2 * p]], axis=1)
                cp = _gelu(jnp.dot(Xp, w0,
                                   preferred_element_type=jnp.float32) + tb)
                t = cp if t is None else jnp.maximum(t, cp)
            h7_ref[1 + m] = _ln_rows(t, tls, tlb)
        acc_ref[:] = jnp.zeros((B, DIM2), jnp.float32)

    ac = aT_ref[pl.ds(i * (DIM2 // CHUNKS), DIM2 // CHUNKS), :]
    b1c = b1c_ref[pl.ds(i * (DIM2 // CHUNKS), DIM2 // CHUNKS), :]
    out1c = (b1c.reshape(1, DIM2 // CHUNKS, 1)
             + h7_ref[0].reshape(B, 1, DIM) * ac[:, 0].reshape(1, -1, 1))
    for n in range(1, 7):
        out1c = out1c + (h7_ref[n].reshape(B, 1, DIM)
                         * ac[:, n].reshape(1, -1, 1))
    out1b = out1c.astype(jnp.bfloat16).reshape(B, CK)
    w2b = w2f_ref[:].astype(jnp.bfloat16).reshape(DIM2, CK)
    acc_ref[:] += jax.lax.dot_general(
        out1b, w2b, (((1,), (1,)), ((), ())),
        preferred_element_type=jnp.float32)

    @pl.when(i == CHUNKS - 1)
    def _tail():
        out2 = acc_ref[:] + ds_b2_ref[:]
        g0 = _gelu(_ln_rows(out2, dls_ref[:], dlb_ref[:]))
        ids = tid_ref[:]
        onehot = (jax.lax.broadcasted_iota(jnp.int32, (B, NTASKS), 1)
                  == ids).astype(jnp.float32)
        t_vec = jnp.dot(onehot, temb_ref[:],
                        preferred_element_type=jnp.float32)
        dims_t = (((1,), (1,)), ((), ()))
        gw = gw_ref[:]
        logits = (jax.lax.dot_general(g0, gw[:, :DIM2], dims_t,
                                      preferred_element_type=jnp.float32)
                  + jax.lax.dot_general(t_vec, gw[:, DIM2:], dims_t,
                                        preferred_element_type=jnp.float32)
                  + gb_ref[:])
        idx = jax.lax.broadcasted_iota(jnp.int32, (B, E), 1)
        m1 = jnp.max(logits, axis=-1, keepdims=True)
        i1 = jnp.min(jnp.where(logits == m1, idx, E), axis=-1, keepdims=True)
        l2 = jnp.where(idx == i1, -jnp.inf, logits)
        m2 = jnp.max(l2, axis=-1, keepdims=True)
        i2 = jnp.min(jnp.where(l2 == m2, idx, E), axis=-1, keepdims=True)
        sel = (idx == i1) | (idx == i2)
        ex = jnp.where(sel, jnp.exp(logits - m1), 0.0)
        gates = ex / jnp.sum(ex, axis=-1, keepdims=True)
        t_task = jnp.zeros((B, DIM2), jnp.float32)
        for e in range(E):
            eo = _gelu(jax.lax.dot_general(
                g0, ewt_ref[e], dims_t, preferred_element_type=jnp.float32)
                + eb_ref[e:e + 1, :])
            t_task = t_task + gates[:, e:e + 1] * eo
        t_univ = _gelu(jax.lax.dot_general(
            g0, uwt_ref[:], dims_t, preferred_element_type=jnp.float32)
            + ub_ref[:])
        omega = 1.0 - jnp.max(gates, axis=-1, keepdims=True)
        out = t_task + omega * t_univ
        out_ref[:] = _ln_rows(out, ols_ref[:], olb_ref[:])


@jax.jit
def kernel(x, tok_conv_w, tok_conv_b, cls_token, tok_ln_s, tok_ln_b,
           ds_w1, ds_b1, ds_w2, ds_b2, ds_ln_s, ds_ln_b,
           task_embed, gate_w, gate_b, expert_w, expert_b,
           univ_w, univ_b, out_ln_s, out_ln_b, task_ids):
    xin = jnp.pad(x[:, :, :2 * (NP - 1) + KC - 6],
                  ((0, 0), (0, 0), (6, 1))).reshape(B, C * 48)
    w0 = jnp.pad(tok_conv_w.reshape(DIM, C, KC).transpose(1, 2, 0),
                 ((0, 0), (0, 48 - KC), (0, 0))).reshape(C * 48, DIM)
    aT = ds_w1[:, 0, 0, 6:6 + 7]
    b1c = ds_b1.reshape(DIM2, 1)
    w2f = ds_w2.reshape(DIM2, DIM2, DIM)
    row = lambda v: v.reshape(1, -1)

    grid = (CHUNKS,)
    full = lambda shape: pl.BlockSpec(shape, lambda i: (0,) * len(shape))
    out = pl.pallas_call(
        _body,
        grid=grid,
        in_specs=[
            full((B, C * 48)),
            full((C * 48, DIM)),
            full((1, DIM)),
            full((1, DIM)),
            full((1, DIM)),
            full((1, DIM)),
            full((DIM2, 7)),
            full((DIM2, 1)),
            pl.BlockSpec((DIM2, DIM2 // CHUNKS, DIM),
                         lambda i: (0, i, 0)),
            full((1, DIM2)),
            full((1, DIM2)),
            full((1, DIM2)),
            full((NTASKS, DIM2)),
            full((B, 1)),
            full((E, 2 * DIM2)),
            full((1, E)),
            full((E, DIM2, DIM2)),
            full((E, DIM2)),
            full((DIM2, DIM2)),
            full((1, DIM2)),
            full((1, DIM2)),
            full((1, DIM2)),
        ],
        out_specs=pl.BlockSpec((B, DIM2), lambda i: (0, 0)),
        out_shape=jax.ShapeDtypeStruct((B, DIM2), jnp.float32),
        scratch_shapes=[
            pltpu.VMEM((7, B, DIM), jnp.float32),
            pltpu.VMEM((B, DIM2), jnp.float32),
        ],
        compiler_params=pltpu.CompilerParams(
            dimension_semantics=("arbitrary",)),
    )(xin, w0, row(tok_conv_b), cls_token[0], row(tok_ln_s), row(tok_ln_b),
      aT, b1c, w2f,
      row(ds_b2), row(ds_ln_s), row(ds_ln_b),
      task_embed, task_ids.reshape(B, 1),
      gate_w, row(gate_b),
      expert_w, expert_b, univ_w, row(univ_b), row(out_ln_s), row(out_ln_b))
    return out


# speedup vs baseline: 1.2076x; 1.0642x over previous
import functools

import jax
import jax.numpy as jnp
import numpy as np
from jax.experimental import pallas as pl
from jax.experimental.pallas import tpu as pltpu

B, C, T = 32, 16, 4096
DIM, DIM2, E, NTASKS, KTOP = 128, 256, 8, 5, 2
NP = 18
NTOK = 6
KC = 13
CHUNKS = 8
CK = (DIM2 * DIM) // CHUNKS
SQRT2 = np.float32(np.sqrt(2.0))


def _gelu(v):
    return v * 0.5 * (1.0 + jax.lax.erf(v / SQRT2))


def _ln_rows(v, s, b, eps=1e-5):
    m = jnp.mean(v, axis=-1, keepdims=True)
    var = jnp.mean((v - m) ** 2, axis=-1, keepdims=True)
    return (v - m) * jax.lax.rsqrt(var + eps) * s + b


def _body(xin_ref, w0_ref, tok_b_ref, cls_ref, tls_ref, tlb_ref,
          aT_ref, b1c_ref, w2f_ref,
          ds_b2_ref, dls_ref, dlb_ref,
          temb_ref, tid_ref, gw_ref, gb_ref,
          ewt_ref, eb_ref, uwt_ref, ub_ref, ols_ref, olb_ref,
          out_ref, h7_ref, acc_ref):
    i = pl.program_id(0)

    @pl.when(i == 0)
    def _front():
        X = jnp.concatenate(
            [jnp.pad(xin_ref[:, c, :42], ((0, 0), (6, 0)))
             for c in range(C)], axis=1)
        w0 = w0_ref[:]
        tb = tok_b_ref[:]
        tls, tlb = tls_ref[:], tlb_ref[:]
        cls_ln = _ln_rows(cls_ref[:], tls, tlb)
        h7_ref[0] = jnp.broadcast_to(cls_ln, (B, DIM))
        for m in range(NTOK):
            t = None
            for p in range(3 * m, 3 * m + 3):
                Xp = X if p == 0 else jnp.concatenate(
                    [X[:, 2 * p:], X[:, :2 * p]], axis=1)
                cp = _gelu(jnp.dot(Xp, w0,
                                   preferred_element_type=jnp.float32) + tb)
                t = cp if t is None else jnp.maximum(t, cp)
            h7_ref[1 + m] = _ln_rows(t, tls, tlb)
        acc_ref[:] = jnp.zeros((B, DIM2), jnp.float32)

    ac = aT_ref[pl.ds(i * (DIM2 // CHUNKS), DIM2 // CHUNKS), 6:13]
    b1c = b1c_ref[pl.ds(i * (DIM2 // CHUNKS), DIM2 // CHUNKS), :]
    out1c = (b1c.reshape(1, DIM2 // CHUNKS, 1)
             + h7_ref[0].reshape(B, 1, DIM) * ac[:, 0].reshape(1, -1, 1))
    for n in range(1, 7):
        out1c = out1c + (h7_ref[n].reshape(B, 1, DIM)
                         * ac[:, n].reshape(1, -1, 1))
    out1b = out1c.astype(jnp.bfloat16).reshape(B, CK)
    w2b = w2f_ref[:].astype(jnp.bfloat16).reshape(DIM2, CK)
    acc_ref[:] += jax.lax.dot_general(
        out1b, w2b, (((1,), (1,)), ((), ())),
        preferred_element_type=jnp.float32)

    @pl.when(i == CHUNKS - 1)
    def _tail():
        out2 = acc_ref[:] + ds_b2_ref[:]
        g0 = _gelu(_ln_rows(out2, dls_ref[:], dlb_ref[:]))
        ids = tid_ref[:]
        onehot = (jax.lax.broadcasted_iota(jnp.int32, (B, NTASKS), 1)
                  == ids).astype(jnp.float32)
        t_vec = jnp.dot(onehot, temb_ref[:],
                        preferred_element_type=jnp.float32)
        dims_t = (((1,), (1,)), ((), ()))
        gw = gw_ref[:]
        logits = (jax.lax.dot_general(g0, gw[:, :DIM2], dims_t,
                                      preferred_element_type=jnp.float32)
                  + jax.lax.dot_general(t_vec, gw[:, DIM2:], dims_t,
                                        preferred_element_type=jnp.float32)
                  + gb_ref[:])
        idx = jax.lax.broadcasted_iota(jnp.int32, (B, E), 1)
        m1 = jnp.max(logits, axis=-1, keepdims=True)
        i1 = jnp.min(jnp.where(logits == m1, idx, E), axis=-1, keepdims=True)
        l2 = jnp.where(idx == i1, -jnp.inf, logits)
        m2 = jnp.max(l2, axis=-1, keepdims=True)
        i2 = jnp.min(jnp.where(l2 == m2, idx, E), axis=-1, keepdims=True)
        sel = (idx == i1) | (idx == i2)
        ex = jnp.where(sel, jnp.exp(logits - m1), 0.0)
        gates = ex / jnp.sum(ex, axis=-1, keepdims=True)
        t_task = jnp.zeros((B, DIM2), jnp.float32)
        for e in range(E):
            eo = _gelu(jax.lax.dot_general(
                g0, ewt_ref[e], dims_t, preferred_element_type=jnp.float32)
                + eb_ref[e:e + 1, :])
            t_task = t_task + gates[:, e:e + 1] * eo
        t_univ = _gelu(jax.lax.dot_general(
            g0, uwt_ref[:], dims_t, preferred_element_type=jnp.float32)
            + ub_ref[:])
        omega = 1.0 - jnp.max(gates, axis=-1, keepdims=True)
        out = t_task + omega * t_univ
        out_ref[:] = _ln_rows(out, ols_ref[:], olb_ref[:])


@jax.jit
def kernel(x, tok_conv_w, tok_conv_b, cls_token, tok_ln_s, tok_ln_b,
           ds_w1, ds_b1, ds_w2, ds_b2, ds_ln_s, ds_ln_b,
           task_embed, gate_w, gate_b, expert_w, expert_b,
           univ_w, univ_b, out_ln_s, out_ln_b, task_ids):
    w0 = jnp.pad(tok_conv_w.reshape(DIM, C, KC).transpose(1, 2, 0),
                 ((0, 0), (0, 48 - KC), (0, 0))).reshape(C * 48, DIM)
    aT = ds_w1.reshape(DIM2, 13)
    b1c = ds_b1.reshape(DIM2, 1)
    w2f = ds_w2.reshape(DIM2, DIM2, DIM)
    row = lambda v: v.reshape(1, -1)

    grid = (CHUNKS,)
    full = lambda shape: pl.BlockSpec(shape, lambda i: (0,) * len(shape))
    out = pl.pallas_call(
        _body,
        grid=grid,
        in_specs=[
            pl.BlockSpec((B, C, DIM), lambda i: (0, 0, 0)),
            full((C * 48, DIM)),
            full((1, DIM)),
            full((1, DIM)),
            full((1, DIM)),
            full((1, DIM)),
            full((DIM2, 13)),
            full((DIM2, 1)),
            pl.BlockSpec((DIM2, DIM2 // CHUNKS, DIM),
                         lambda i: (0, i, 0)),
            full((1, DIM2)),
            full((1, DIM2)),
            full((1, DIM2)),
            full((NTASKS, DIM2)),
            full((B, 1)),
            full((E, 2 * DIM2)),
            full((1, E)),
            full((E, DIM2, DIM2)),
            full((E, DIM2)),
            full((DIM2, DIM2)),
            full((1, DIM2)),
            full((1, DIM2)),
            full((1, DIM2)),
        ],
        out_specs=pl.BlockSpec((B, DIM2), lambda i: (0, 0)),
        out_shape=jax.ShapeDtypeStruct((B, DIM2), jnp.float32),
        scratch_shapes=[
            pltpu.VMEM((7, B, DIM), jnp.float32),
            pltpu.VMEM((B, DIM2), jnp.float32),
        ],
        compiler_params=pltpu.CompilerParams(
            dimension_semantics=("arbitrary",)),
    )(x, w0, row(tok_conv_b), cls_token[0], row(tok_ln_s), row(tok_ln_b),
      aT, b1c, w2f,
      row(ds_b2), row(ds_ln_s), row(ds_ln_b),
      task_embed, task_ids.reshape(B, 1),
      gate_w, row(gate_b),
      expert_w, expert_b, univ_w, row(univ_b), row(out_ln_s), row(out_ln_b))
    return out
